# baseline (speedup 1.0000x reference)
import jax
import jax.numpy as jnp
from jax import lax
from jax.experimental import pallas as pl
from jax.experimental.pallas import tpu as pltpu
from jax.experimental.pallas import tpu_sc as plsc

E = 16
D = 1024
H = 128
K = 8
CH = 32
N = 2048

_PREC = lax.Precision.DEFAULT

_NC = 2
_NS = 16
_NW = _NC * _NS
_RPW = N // _NW


def _gate_body(x_ref, wg1_ref, bg1_ref, wg2_ref, bg2_ref, w_ref, sp_ref):
    x = x_ref[...]
    h = jnp.maximum(jnp.dot(x, wg1_ref[...], precision=_PREC) + bg1_ref[...], 0.0)
    logits = jnp.dot(h, wg2_ref[...], precision=_PREC) + bg2_ref[...]
    m = jnp.max(logits, axis=1, keepdims=True)
    ex = jnp.exp(logits - m)
    w = ex / jnp.sum(ex, axis=1, keepdims=True)
    w_ref[...] = w
    sp_ref[...] = jnp.sum((w > 0.01).astype(jnp.float32)).reshape(1, 1) / (N * E)


_gating = pl.pallas_call(
    _gate_body,
    out_shape=[
        jax.ShapeDtypeStruct((N, E), jnp.float32),
        jax.ShapeDtypeStruct((1, 1), jnp.float32),
    ],
)


def _route_body(w_hbm, ti_hbm, wm_hbm, w_v, ti_v, wm_v):
    wid = lax.axis_index("s") * _NC + lax.axis_index("c")
    base = wid * _RPW
    pltpu.sync_copy(w_hbm.at[pl.ds(base, _RPW)], w_v)
    lanes = lax.broadcasted_iota(jnp.int32, (E,), 0)
    topmask = lanes < K
    for i in range(_RPW):
        w = w_v[i, :]
        sk, si = plsc.sort_key_val(w, lanes, descending=True)
        ti_v[i, :] = si
        mv = jnp.where(topmask, sk, 0.0)
        row = jnp.full((E,), i, dtype=jnp.int32)
        plsc.store_scatter(wm_v, [row, si], mv)
    pltpu.sync_copy(ti_v, ti_hbm.at[pl.ds(base, _RPW)])
    pltpu.sync_copy(wm_v, wm_hbm.at[pl.ds(base, _RPW)])


_route_cached = None


def _route(weights):
    global _route_cached
    if _route_cached is None:
        _route_cached = pl.kernel(
            _route_body,
            out_type=(
                jax.ShapeDtypeStruct((N, E), jnp.int32),
                jax.ShapeDtypeStruct((N, E), jnp.float32),
            ),
            mesh=plsc.VectorSubcoreMesh(core_axis_name="c", subcore_axis_name="s"),
            scratch_types=[
                pltpu.VMEM((_RPW, E), jnp.float32),
                pltpu.VMEM((_RPW, E), jnp.int32),
                pltpu.VMEM((_RPW, E), jnp.float32),
            ],
            compiler_params=pltpu.CompilerParams(needs_layout_passes=False),
        )
    return _route_cached(weights)


def _expert_body(x_ref, wm_ref, ti_ref, w1_ref, b1_ref, w2_ref, b2_ref,
                 w3_ref, b3_ref, wc1_ref, bc1_ref, wc2_ref,
                 out_ref, csel_ref, h2all, conf_acc):
    e = pl.program_id(0)
    h1 = jnp.maximum(jnp.dot(x_ref[...], w1_ref[0], precision=_PREC,
                             preferred_element_type=jnp.float32) + b1_ref[0], 0.0)
    h1b = h1.astype(jnp.bfloat16)
    h2 = jnp.maximum(jnp.dot(h1b, w2_ref[0], precision=_PREC,
                             preferred_element_type=jnp.float32) + b2_ref[0], 0.0)
    wc1 = wc1_ref[0]
    v = jnp.dot(w3_ref[e], wc1, precision=_PREC,
                preferred_element_type=jnp.float32)
    u = jnp.dot(b3_ref[e], wc1, precision=_PREC)
    c1 = jnp.maximum(jnp.dot(h2, v, precision=_PREC) + u + bc1_ref[0], 0.0)
    c1a = jnp.concatenate([c1, jnp.ones((N, 1), jnp.float32)], axis=1)
    logit = lax.dot_general(c1a, wc2_ref[0], (((1,), (1,)), ((), ())),
                            precision=_PREC)
    conf = jax.nn.sigmoid(logit)
    onehot = (lax.broadcasted_iota(jnp.int32, (1, E), 1) == e).astype(jnp.float32)
    wcol = lax.dot_general(wm_ref[...], onehot, (((1,), (1,)), ((), ())),
                           precision=_PREC)
    g = wcol * conf
    h2all[:, pl.ds(e * H, H)] = (g * h2).astype(jnp.bfloat16)

    @pl.when(e == 0)
    def _():
        conf_acc[...] = conf * onehot

    @pl.when(e > 0)
    def _():
        conf_acc[...] = conf_acc[...] + conf * onehot

    @pl.when(e == E - 1)
    def _():
        ca = conf_acc[...]
        gm = wm_ref[...] * ca
        s = jnp.sum(gm, axis=1, keepdims=True)
        w3r = w3_ref[...].reshape(E * H, D)
        acc = jnp.dot(h2all[...], w3r, precision=_PREC,
                      preferred_element_type=jnp.float32)
        acc = acc + jnp.dot(gm, b3_ref[...].reshape(E, D), precision=_PREC)
        out_ref[...] = acc / (s + 1e-8)
        iota_row = lax.broadcasted_iota(jnp.int32, (1, E), 1)
        ti = ti_ref[...]
        cols = []
        for k in range(K):
            eqk = ti[:, k:k + 1] == iota_row
            cols.append(jnp.sum(jnp.where(eqk, ca, 0.0), axis=1, keepdims=True))
        csel_ref[...] = jnp.concatenate(cols, axis=1)


_experts = pl.pallas_call(
    _expert_body,
    grid=(E,),
    in_specs=[
        pl.BlockSpec((N, D), lambda e: (0, 0)),
        pl.BlockSpec((N, E), lambda e: (0, 0)),
        pl.BlockSpec((N, E), lambda e: (0, 0)),
        pl.BlockSpec((1, D, H), lambda e: (e, 0, 0)),
        pl.BlockSpec((1, 1, H), lambda e: (e, 0, 0)),
        pl.BlockSpec((1, H, H), lambda e: (e, 0, 0)),
        pl.BlockSpec((1, 1, H), lambda e: (e, 0, 0)),
        pl.BlockSpec((E, H, D), lambda e: (0, 0, 0)),
        pl.BlockSpec((E, 1, D), lambda e: (0, 0, 0)),
        pl.BlockSpec((1, D, CH), lambda e: (e, 0, 0)),
        pl.BlockSpec((1, 1, CH), lambda e: (e, 0, 0)),
        pl.BlockSpec((1, 1, CH + 1), lambda e: (e, 0, 0)),
    ],
    out_specs=[
        pl.BlockSpec((N, D), lambda e: (0, 0)),
        pl.BlockSpec((N, K), lambda e: (0, 0)),
    ],
    out_shape=[
        jax.ShapeDtypeStruct((N, D), jnp.float32),
        jax.ShapeDtypeStruct((N, K), jnp.float32),
    ],
    scratch_shapes=[
        pltpu.VMEM((N, E * H), jnp.bfloat16),
        pltpu.VMEM((N, E), jnp.float32),
    ],
    compiler_params=pltpu.CompilerParams(
        dimension_semantics=("arbitrary",),
        vmem_limit_bytes=100 * 1024 * 1024,
    ),
)


def kernel(x, W1, b1, W2, b2, W3, b3, Wg1, bg1, Wg2, bg2, Wc1, bc1, Wc2, bc2):
    weights, sp = _gating(x, Wg1, bg1.reshape(1, H), Wg2, bg2.reshape(1, E))

    ti, wm = _route(weights)
    wc2b = jnp.concatenate([Wc2[:, :, 0], bc2], axis=1).reshape(E, 1, CH + 1)
    out, csel = _experts(x.astype(jnp.bfloat16), wm, ti,
                         W1, b1.reshape(E, 1, H), W2,
                         b2.reshape(E, 1, H), W3.astype(jnp.bfloat16),
                         b3.reshape(E, 1, D),
                         Wc1, bc1.reshape(E, 1, CH), wc2b)
    return out, weights, csel, sp.reshape(())


# speedup vs baseline: 1.2175x; 1.2175x over previous
import jax
import jax.numpy as jnp
from jax import lax
from jax.experimental import pallas as pl
from jax.experimental.pallas import tpu as pltpu
from jax.experimental.pallas import tpu_sc as plsc

E = 16
D = 1024
H = 128
K = 8
CH = 32
N = 2048

_PREC = lax.Precision.DEFAULT

_NC = 2
_NS = 16
_NW = _NC * _NS
_RPW = N // _NW


def _gate_body(x_ref, wg1_ref, bg1_ref, wg2_ref, bg2_ref, w_ref, sp_ref):
    x = x_ref[...]
    h = jnp.maximum(jnp.dot(x, wg1_ref[...], precision=_PREC) + bg1_ref[...], 0.0)
    logits = jnp.dot(h, wg2_ref[...], precision=_PREC) + bg2_ref[...]
    m = jnp.max(logits, axis=1, keepdims=True)
    ex = jnp.exp(logits - m)
    w = ex / jnp.sum(ex, axis=1, keepdims=True)
    w_ref[...] = w
    sp_ref[...] = jnp.sum((w > 0.01).astype(jnp.float32)).reshape(1, 1) / (N * E)


_gating = pl.pallas_call(
    _gate_body,
    out_shape=[
        jax.ShapeDtypeStruct((N, E), jnp.float32),
        jax.ShapeDtypeStruct((1, 1), jnp.float32),
    ],
)


def _route_body(w_hbm, ti_hbm, wm_hbm, w_v, ti_v, wm_v):
    wid = lax.axis_index("s") * _NC + lax.axis_index("c")
    base = wid * _RPW
    pltpu.sync_copy(w_hbm.at[pl.ds(base, _RPW)], w_v)
    lanes = lax.broadcasted_iota(jnp.int32, (E,), 0)
    topmask = lanes < K
    for i in range(_RPW):
        w = w_v[i, :]
        sk, si = plsc.sort_key_val(w, lanes, descending=True)
        ti_v[i, :] = si
        mv = jnp.where(topmask, sk, 0.0)
        row = jnp.full((E,), i, dtype=jnp.int32)
        plsc.store_scatter(wm_v, [row, si], mv)
    pltpu.sync_copy(ti_v, ti_hbm.at[pl.ds(base, _RPW)])
    pltpu.sync_copy(wm_v, wm_hbm.at[pl.ds(base, _RPW)])


_route_cached = None


def _route(weights):
    global _route_cached
    if _route_cached is None:
        _route_cached = pl.kernel(
            _route_body,
            out_type=(
                jax.ShapeDtypeStruct((N, E), jnp.int32),
                jax.ShapeDtypeStruct((N, E), jnp.float32),
            ),
            mesh=plsc.VectorSubcoreMesh(core_axis_name="c", subcore_axis_name="s"),
            scratch_types=[
                pltpu.VMEM((_RPW, E), jnp.float32),
                pltpu.VMEM((_RPW, E), jnp.int32),
                pltpu.VMEM((_RPW, E), jnp.float32),
            ],
            compiler_params=pltpu.CompilerParams(needs_layout_passes=False),
        )
    return _route_cached(weights)


def _expert_body(x_ref, wm_ref, ti_ref, w1_ref, b1_ref, w2_ref, b2_ref,
                 w3_ref, b3_ref, wc1_ref, bc1_ref, wc2_ref,
                 out_ref, csel_ref, h2all, conf_acc):
    i = pl.program_id(0)
    w1cat = jnp.concatenate([w1_ref[0], w1_ref[1]], axis=1)
    b1cat = jnp.concatenate([b1_ref[0], b1_ref[1]], axis=1)
    h1 = jnp.maximum(jnp.dot(x_ref[...], w1cat, precision=_PREC) + b1cat, 0.0)
    wc1_both = wc1_ref[...]
    iota_row_f = lax.broadcasted_iota(jnp.int32, (1, E), 1)
    conf_upd = None
    for j in range(2):
        e = 2 * i + j
        h2 = jnp.maximum(jnp.dot(h1[:, j * H:(j + 1) * H], w2_ref[j],
                                 precision=_PREC) + b2_ref[j], 0.0)
        wc1 = wc1_both[j]
        v = jnp.dot(w3_ref[e], wc1, precision=_PREC)
        u = jnp.dot(b3_ref[e], wc1, precision=_PREC)
        c1 = jnp.maximum(jnp.dot(h2, v, precision=_PREC) + u + bc1_ref[j], 0.0)
        c1a = jnp.concatenate([c1, jnp.ones((N, 1), jnp.float32)], axis=1)
        logit = lax.dot_general(c1a, wc2_ref[j], (((1,), (1,)), ((), ())),
                                precision=_PREC)
        conf = jax.nn.sigmoid(logit)
        onehot = (iota_row_f == e).astype(jnp.float32)
        wcol = lax.dot_general(wm_ref[...], onehot, (((1,), (1,)), ((), ())),
                               precision=_PREC)
        g = wcol * conf
        contrib = conf * onehot
        conf_upd = contrib if conf_upd is None else conf_upd + contrib
        h2all[:, pl.ds(e * H, H)] = (g * h2).astype(jnp.bfloat16)

    @pl.when(i == 0)
    def _():
        conf_acc[...] = conf_upd

    @pl.when(i > 0)
    def _():
        conf_acc[...] = conf_acc[...] + conf_upd

    @pl.when(i == E // 2 - 1)
    def _():
        ca = conf_acc[...]
        gm = wm_ref[...] * ca
        s = jnp.sum(gm, axis=1, keepdims=True)
        w3r = w3_ref[...].reshape(E * H, D)
        acc = jnp.dot(h2all[...], w3r, precision=_PREC,
                      preferred_element_type=jnp.float32)
        acc = acc + jnp.dot(gm, b3_ref[...].reshape(E, D), precision=_PREC)
        out_ref[...] = acc / (s + 1e-8)
        iota_row = lax.broadcasted_iota(jnp.int32, (1, E), 1)
        ti = ti_ref[...]
        cols = []
        for k in range(K):
            eqk = ti[:, k:k + 1] == iota_row
            cols.append(jnp.sum(jnp.where(eqk, ca, 0.0), axis=1, keepdims=True))
        csel_ref[...] = jnp.concatenate(cols, axis=1)


_experts = pl.pallas_call(
    _expert_body,
    grid=(E // 2,),
    in_specs=[
        pl.BlockSpec((N, D), lambda i: (0, 0)),
        pl.BlockSpec((N, E), lambda i: (0, 0)),
        pl.BlockSpec((N, E), lambda i: (0, 0)),
        pl.BlockSpec((2, D, H), lambda i: (i, 0, 0)),
        pl.BlockSpec((2, 1, H), lambda i: (i, 0, 0)),
        pl.BlockSpec((2, H, H), lambda i: (i, 0, 0)),
        pl.BlockSpec((2, 1, H), lambda i: (i, 0, 0)),
        pl.BlockSpec((E, H, D), lambda i: (0, 0, 0)),
        pl.BlockSpec((E, 1, D), lambda i: (0, 0, 0)),
        pl.BlockSpec((2, D, CH), lambda i: (i, 0, 0)),
        pl.BlockSpec((2, 1, CH), lambda i: (i, 0, 0)),
        pl.BlockSpec((2, 1, CH + 1), lambda i: (i, 0, 0)),
    ],
    out_specs=[
        pl.BlockSpec((N, D), lambda i: (0, 0)),
        pl.BlockSpec((N, K), lambda i: (0, 0)),
    ],
    out_shape=[
        jax.ShapeDtypeStruct((N, D), jnp.float32),
        jax.ShapeDtypeStruct((N, K), jnp.float32),
    ],
    scratch_shapes=[
        pltpu.VMEM((N, E * H), jnp.bfloat16),
        pltpu.VMEM((N, E), jnp.float32),
    ],
    compiler_params=pltpu.CompilerParams(
        dimension_semantics=("arbitrary",),
        vmem_limit_bytes=100 * 1024 * 1024,
    ),
)


def kernel(x, W1, b1, W2, b2, W3, b3, Wg1, bg1, Wg2, bg2, Wc1, bc1, Wc2, bc2):
    weights, sp = _gating(x, Wg1, bg1.reshape(1, H), Wg2, bg2.reshape(1, E))

    ti, wm = _route(weights)
    wc2b = jnp.concatenate([Wc2[:, :, 0], bc2], axis=1).reshape(E, 1, CH + 1)
    out, csel = _experts(x, wm, ti, W1, b1.reshape(E, 1, H), W2,
                         b2.reshape(E, 1, H), W3, b3.reshape(E, 1, D),
                         Wc1, bc1.reshape(E, 1, CH), wc2b)
    return out, weights, csel, sp.reshape(())
